# baseline (speedup 1.0000x reference)
import functools

import jax
import jax.numpy as jnp
from jax import lax
from jax.experimental import pallas as pl
from jax.experimental.pallas import tpu as pltpu
from jax.experimental.pallas import tpu_sc as plsc

BATCH = 16384
EMBED_DIM = 64
PAD_DIM = 128
LANES = 16

_info = plsc.get_sparse_core_info()
_NC, _NS = _info.num_cores, _info.num_subcores
_NW = _NC * _NS
_B_PER_W = BATCH // _NW
_CHUNK = 128
_NCH = _B_PER_W // _CHUNK


def _lookup_body(labels_hbm, table_hbm, out_hbm, idx_v, g_v, r_v, gsem, wsem):
    wid = lax.axis_index("s") * _NC + lax.axis_index("c")
    base = wid * _B_PER_W
    pltpu.sync_copy(labels_hbm.at[pl.ds(base, _B_PER_W)], idx_v)

    def gather_chunk(t):
        return pltpu.async_copy(
            table_hbm.at[idx_v.at[pl.ds(t * _CHUNK, _CHUNK)]],
            g_v.at[t % 2],
            gsem[t % 2],
        )

    def repack_chunk(t):
        def row(i, _):
            for j in range(EMBED_DIM // LANES):
                r_v[t % 2, i, pl.ds(j * LANES, LANES)] = g_v[
                    t % 2, i, pl.ds(j * LANES, LANES)
                ]
            return 0

        lax.fori_loop(0, _CHUNK, row, 0, unroll=2)

    def write_chunk(t):
        return pltpu.async_copy(
            r_v.at[t % 2],
            out_hbm.at[pl.ds(base + t * _CHUNK, _CHUNK)],
            wsem[t % 2],
        )

    writes = [None, None]
    g = gather_chunk(0)
    for t in range(_NCH):
        g.wait()
        if t + 1 < _NCH:
            g_next = gather_chunk(t + 1)
        if writes[t % 2] is not None:
            writes[t % 2].wait()
        repack_chunk(t)
        writes[t % 2] = write_chunk(t)
        if t + 1 < _NCH:
            g = g_next
    writes[(_NCH - 1) % 2].wait()
    writes[_NCH % 2].wait()


@jax.jit
def kernel(labels, table):
    table_pad = jnp.pad(table, ((0, 0), (0, PAD_DIM - EMBED_DIM)))
    k = functools.partial(
        pl.kernel,
        mesh=plsc.VectorSubcoreMesh(core_axis_name="c", subcore_axis_name="s"),
        out_type=jax.ShapeDtypeStruct((BATCH, EMBED_DIM), jnp.float32),
        scratch_types=[
            pltpu.VMEM((_B_PER_W,), jnp.int32),
            pltpu.VMEM((2, _CHUNK, PAD_DIM), jnp.float32),
            pltpu.VMEM((2, _CHUNK, EMBED_DIM), jnp.float32),
            [pltpu.SemaphoreType.DMA, pltpu.SemaphoreType.DMA],
            [pltpu.SemaphoreType.DMA, pltpu.SemaphoreType.DMA],
        ],
        compiler_params=pltpu.CompilerParams(use_tc_tiling_on_sc=True),
    )(_lookup_body)
    return k(labels, table_pad)
